# baseline (speedup 1.0000x reference)
import functools

import jax
import jax.numpy as jnp
from jax import lax
from jax.experimental import pallas as pl
from jax.experimental.pallas import tpu as pltpu
from jax.experimental.pallas import tpu_sc as plsc

_N = 10000
_D = 128
_K = 8
_DD = 16
_ROUTIT = 6
_NLAYER = 3
_NCLASS = 40

_NROWS = 10496
_HALF = 5120
_SENT = 10240
_M = 320000
_CH = 128
_NT = 16
_NCHUNK = 157
_MPAD = _NCHUNK * _CH * _NT
_AGG_ROWS = _HALF + 64
_ROWS_PT = _HALF // _NT


def _rsqrt_nr(ss):
    bi = plsc.bitcast(ss, jnp.int32)
    y = plsc.bitcast(jnp.int32(0x5F3759DF) - (bi >> 1), jnp.float32)
    y = y * (1.5 - 0.5 * ss * y * y)
    y = y * (1.5 - 0.5 * ss * y * y)
    y = y * (1.5 - 0.5 * ss * y * y)
    return y


def _route_body(xn, c, srcp, trgp, c_new, zbuf, cbuf, isrc, itrg, iloc,
                agg, sem):
    cid = lax.axis_index("c")
    sid = lax.axis_index("s")
    my_lo = cid * _HALF
    lane = lax.iota(jnp.int32, _DD)
    cols = [jnp.full((_DD,), kj, jnp.int32) for kj in range(_D)]

    def _zero_zbuf(i, _):
        for k in range(_K):
            zbuf[i, pl.ds(k * _DD, _DD)] = jnp.zeros((_DD,), jnp.float32)
        return 0
    lax.fori_loop(0, _CH, _zero_zbuf, 0)

    rpt = _AGG_ROWS // _NT
    abase = sid * rpt
    pltpu.sync_copy(zbuf, agg.at[pl.ds(abase, _CH)])
    pltpu.sync_copy(zbuf, agg.at[pl.ds(abase + _CH, _CH)])
    pltpu.sync_copy(zbuf.at[pl.ds(0, rpt - 2 * _CH)],
                    agg.at[pl.ds(abase + 2 * _CH, rpt - 2 * _CH)])
    plsc.subcore_barrier()

    def _chunk(ch, _):
        ebase = (sid * _NCHUNK + ch) * _CH
        pltpu.sync_copy(srcp.at[pl.ds(ebase, _CH)], isrc)
        pltpu.sync_copy(trgp.at[pl.ds(ebase, _CH)], itrg)
        pltpu.async_copy(xn.at[isrc], zbuf, sem).wait()
        pltpu.async_copy(c.at[itrg], cbuf, sem).wait()

        def _grp(g, _):
            rows = g * _DD + lane
            p = []
            for k in range(_K):
                acc = None
                for j in range(_DD):
                    col = cols[k * _DD + j]
                    zv = plsc.load_gather(zbuf, [rows, col])
                    cv = plsc.load_gather(cbuf, [rows, col])
                    acc = zv * cv if acc is None else acc + zv * cv
                p.append(acc)
            m = p[0]
            for k in range(1, _K):
                m = jnp.maximum(m, p[k])
            ev = [jnp.exp(p[k] - m) for k in range(_K)]
            s = ev[0]
            for k in range(1, _K):
                s = s + ev[k]
            r = 1.0 / s
            for k in range(_K):
                w = ev[k] * r
                for j in range(_DD):
                    col = cols[k * _DD + j]
                    zv = plsc.load_gather(zbuf, [rows, col])
                    plsc.store_scatter(zbuf, [rows, col], zv * w)
            return 0
        lax.fori_loop(0, _CH // _DD, _grp, 0)

        for g in range(_CH // _DD):
            sl = pl.ds(g * _DD, _DD)
            loc = itrg[sl] - my_lo
            bad = (loc < 0) | (loc >= _HALF)
            iloc[sl] = jnp.where(bad, _HALF, loc)
        pltpu.sync_copy(zbuf, agg.at[iloc], add=True)
        return 0
    lax.fori_loop(0, _NCHUNK, _chunk, 0)
    plsc.subcore_barrier()

    nb = sid * _ROWS_PT
    for off, nr in ((0, _CH), (_CH, _CH), (2 * _CH, _ROWS_PT - 2 * _CH)):
        pltpu.sync_copy(agg.at[pl.ds(nb + off, nr)], cbuf.at[pl.ds(0, nr)])
        pltpu.sync_copy(xn.at[pl.ds(my_lo + nb + off, nr)],
                        zbuf.at[pl.ds(0, nr)])

        def _fin(g, _):
            rows = g * _DD + lane
            for k in range(_K):
                acc = None
                tk = []
                for j in range(_DD):
                    col = cols[k * _DD + j]
                    t = (plsc.load_gather(zbuf, [rows, col]) +
                         plsc.load_gather(cbuf, [rows, col]))
                    tk.append(t)
                    acc = t * t if acc is None else acc + t * t
                y = _rsqrt_nr(jnp.maximum(acc, 1e-24))
                for j in range(_DD):
                    plsc.store_scatter(zbuf, [rows, cols[k * _DD + j]],
                                       tk[j] * y)
            return 0
        lax.fori_loop(0, nr // _DD, _fin, 0)
        pltpu.sync_copy(zbuf.at[pl.ds(0, nr)],
                        c_new.at[pl.ds(my_lo + nb + off, nr)])


_route = pl.kernel(
    _route_body,
    out_type=jax.ShapeDtypeStruct((_NROWS, _D), jnp.float32),
    mesh=plsc.VectorSubcoreMesh(core_axis_name="c", subcore_axis_name="s"),
    compiler_params=pltpu.CompilerParams(needs_layout_passes=False),
    scratch_types=[
        pltpu.VMEM((_CH, _D), jnp.float32),
        pltpu.VMEM((_CH, _D), jnp.float32),
        pltpu.VMEM((_CH,), jnp.int32),
        pltpu.VMEM((_CH,), jnp.int32),
        pltpu.VMEM((_CH,), jnp.int32),
        pltpu.VMEM_SHARED((_AGG_ROWS, _D), jnp.float32),
        pltpu.SemaphoreType.DMA,
    ],
)


def _pca_body(f_ref, w_ref, b_ref, bs_ref, bst_ref, o_ref):
    x = jnp.dot(f_ref[...], w_ref[...], preferred_element_type=jnp.float32)
    x = jnp.maximum(x + b_ref[...], 0.0)
    ss = jnp.dot(x * x, bs_ref[...], preferred_element_type=jnp.float32)
    rs = lax.rsqrt(jnp.maximum(ss, 1e-24))
    o_ref[...] = x * jnp.dot(rs, bst_ref[...],
                             preferred_element_type=jnp.float32)


_pca = pl.pallas_call(
    _pca_body,
    grid=(_NROWS // 256,),
    in_specs=[
        pl.BlockSpec((256, _D), lambda i: (i, 0)),
        pl.BlockSpec((_D, _D), lambda i: (0, 0)),
        pl.BlockSpec((1, _D), lambda i: (0, 0)),
        pl.BlockSpec((_D, _K), lambda i: (0, 0)),
        pl.BlockSpec((_K, _D), lambda i: (0, 0)),
    ],
    out_specs=pl.BlockSpec((256, _D), lambda i: (i, 0)),
    out_shape=jax.ShapeDtypeStruct((_NROWS, _D), jnp.float32),
)


def _mlp_body(x_ref, w_ref, b_ref, o_ref):
    l = jnp.dot(x_ref[...], w_ref[...], preferred_element_type=jnp.float32)
    l = l + b_ref[...]
    m = jnp.max(l, axis=1, keepdims=True)
    e = jnp.exp(l - m)
    o_ref[...] = e / jnp.sum(e, axis=1, keepdims=True)


_mlp = pl.pallas_call(
    _mlp_body,
    grid=(_NROWS // 256,),
    in_specs=[
        pl.BlockSpec((256, _D), lambda i: (i, 0)),
        pl.BlockSpec((_D, _D), lambda i: (0, 0)),
        pl.BlockSpec((1, _D), lambda i: (0, 0)),
    ],
    out_specs=pl.BlockSpec((256, _D), lambda i: (i, 0)),
    out_shape=jax.ShapeDtypeStruct((_NROWS, _D), jnp.float32),
)


@jax.jit
def _run(feat, src, trg, pca_w, pca_b, mlp_w, mlp_b):
    featp = jnp.zeros((_NROWS, _D), jnp.float32).at[:_N].set(feat)
    srcp = jnp.concatenate(
        [src.astype(jnp.int32), jnp.zeros((_MPAD - _M,), jnp.int32)])
    trgp = jnp.concatenate(
        [trg.astype(jnp.int32), jnp.full((_MPAD - _M,), _SENT, jnp.int32)])
    cap = jnp.arange(_D, dtype=jnp.int32) // _DD
    bs = (cap[:, None] == jnp.arange(_K)[None, :]).astype(jnp.float32)
    wp = jnp.zeros((_D, _D), jnp.float32).at[:, :_NCLASS].set(mlp_w)
    bp = jnp.full((_D,), -1e30, jnp.float32).at[:_NCLASS].set(mlp_b)

    x = _pca(featp, pca_w, pca_b.reshape(1, _D), bs, bs.T)
    for _layer in range(_NLAYER):
        cc = x
        for _it in range(_ROUTIT):
            cc = _route(x, cc, srcp, trgp)
        x = cc
    probs = _mlp(x, wp, bp.reshape(1, _D))
    return probs[:_N, :_NCLASS]


def kernel(feat, src_trg_edges, pca_w, pca_b, mlp_w, mlp_b):
    return _run(feat, src_trg_edges[0], src_trg_edges[1],
                pca_w, pca_b, mlp_w, mlp_b)


# speedup vs baseline: 4.2520x; 4.2520x over previous
import functools

import jax
import jax.numpy as jnp
from jax import lax
from jax.experimental import pallas as pl
from jax.experimental.pallas import tpu as pltpu
from jax.experimental.pallas import tpu_sc as plsc

_N = 10000
_D = 128
_K = 8
_DD = 16
_ROUTIT = 6
_NLAYER = 3
_NCLASS = 40

_NROWS = 10496
_HALF = 5120
_SENT = 10240
_M = 320000
_CH = 128
_NT = 16
_MPAD = 323584
_NCHUNK_G = _MPAD // (_CH * 2 * _NT)
_NCHUNK_S = _MPAD // (_CH * _NT)
_AGG_ROWS = _HALF + 64
_ROWS_PT = _HALF // _NT

_SC_PARAMS = pltpu.CompilerParams(needs_layout_passes=False)
_SC_MESH = plsc.VectorSubcoreMesh(core_axis_name="c", subcore_axis_name="s")


def _rsqrt_nr(ss):
    bi = plsc.bitcast(ss, jnp.int32)
    y = plsc.bitcast(jnp.int32(0x5F3759DF) - (bi >> 1), jnp.float32)
    y = y * (1.5 - 0.5 * ss * y * y)
    y = y * (1.5 - 0.5 * ss * y * y)
    y = y * (1.5 - 0.5 * ss * y * y)
    return y


def _gath_body(tab, idx, out, buf, ibuf, sem):
    wid = lax.axis_index("s") * 2 + lax.axis_index("c")

    def _ch(i, _):
        base = (wid * _NCHUNK_G + i) * _CH
        pltpu.sync_copy(idx.at[pl.ds(base, _CH)], ibuf)
        pltpu.async_copy(tab.at[ibuf], buf, sem).wait()
        pltpu.sync_copy(buf, out.at[pl.ds(base, _CH)])
        return 0
    lax.fori_loop(0, _NCHUNK_G, _ch, 0)


_gath = pl.kernel(
    _gath_body,
    out_type=jax.ShapeDtypeStruct((_MPAD, _D), jnp.float32),
    mesh=_SC_MESH,
    compiler_params=_SC_PARAMS,
    scratch_types=[
        pltpu.VMEM((_CH, _D), jnp.float32),
        pltpu.VMEM((_CH,), jnp.int32),
        pltpu.SemaphoreType.DMA,
    ],
)


def _scat_body(wz, trgp, xn, c_new, zbuf, cbuf, itrg, iloc, agg, sem):
    cid = lax.axis_index("c")
    sid = lax.axis_index("s")
    my_lo = cid * _HALF
    lane = lax.iota(jnp.int32, _DD)
    cols = [jnp.full((_DD,), kj, jnp.int32) for kj in range(_D)]

    def _zero_zbuf(i, _):
        for k in range(_K):
            zbuf[i, pl.ds(k * _DD, _DD)] = jnp.zeros((_DD,), jnp.float32)
        return 0
    lax.fori_loop(0, _CH, _zero_zbuf, 0)

    rpt = _AGG_ROWS // _NT
    abase = sid * rpt
    pltpu.sync_copy(zbuf, agg.at[pl.ds(abase, _CH)])
    pltpu.sync_copy(zbuf, agg.at[pl.ds(abase + _CH, _CH)])
    pltpu.sync_copy(zbuf.at[pl.ds(0, rpt - 2 * _CH)],
                    agg.at[pl.ds(abase + 2 * _CH, rpt - 2 * _CH)])
    plsc.subcore_barrier()

    def _chunk(ch, _):
        base = (sid * _NCHUNK_S + ch) * _CH
        pltpu.sync_copy(trgp.at[pl.ds(base, _CH)], itrg)
        pltpu.sync_copy(wz.at[pl.ds(base, _CH)], zbuf)
        for g in range(_CH // _DD):
            sl = pl.ds(g * _DD, _DD)
            loc = itrg[sl] - my_lo
            bad = (loc < 0) | (loc >= _HALF)
            iloc[sl] = jnp.where(bad, _HALF, loc)
        pltpu.sync_copy(zbuf, agg.at[iloc], add=True)
        return 0
    lax.fori_loop(0, _NCHUNK_S, _chunk, 0)
    plsc.subcore_barrier()

    nb = sid * _ROWS_PT
    for off, nr in ((0, _CH), (_CH, _CH), (2 * _CH, _ROWS_PT - 2 * _CH)):
        pltpu.sync_copy(agg.at[pl.ds(nb + off, nr)], cbuf.at[pl.ds(0, nr)])
        pltpu.sync_copy(xn.at[pl.ds(my_lo + nb + off, nr)],
                        zbuf.at[pl.ds(0, nr)])

        def _fin(g, _):
            rows = g * _DD + lane
            for k in range(_K):
                acc = None
                tk = []
                for j in range(_DD):
                    col = cols[k * _DD + j]
                    t = (plsc.load_gather(zbuf, [rows, col]) +
                         plsc.load_gather(cbuf, [rows, col]))
                    tk.append(t)
                    acc = t * t if acc is None else acc + t * t
                y = _rsqrt_nr(jnp.maximum(acc, 1e-24))
                for j in range(_DD):
                    plsc.store_scatter(zbuf, [rows, cols[k * _DD + j]],
                                       tk[j] * y)
            return 0
        lax.fori_loop(0, nr // _DD, _fin, 0)
        pltpu.sync_copy(zbuf.at[pl.ds(0, nr)],
                        c_new.at[pl.ds(my_lo + nb + off, nr)])


_scat = pl.kernel(
    _scat_body,
    out_type=jax.ShapeDtypeStruct((_NROWS, _D), jnp.float32),
    mesh=_SC_MESH,
    compiler_params=_SC_PARAMS,
    scratch_types=[
        pltpu.VMEM((_CH, _D), jnp.float32),
        pltpu.VMEM((_CH, _D), jnp.float32),
        pltpu.VMEM((_CH,), jnp.int32),
        pltpu.VMEM((_CH,), jnp.int32),
        pltpu.VMEM_SHARED((_AGG_ROWS, _D), jnp.float32),
        pltpu.SemaphoreType.DMA,
    ],
)


def _fused_body(z_ref, cg_ref, bs_ref, bst_ref, o_ref):
    z = z_ref[...]
    p = jnp.dot(z * cg_ref[...], bs_ref[...],
                preferred_element_type=jnp.float32)
    m = jnp.max(p, axis=1, keepdims=True)
    e = jnp.exp(p - m)
    w = e / jnp.sum(e, axis=1, keepdims=True)
    o_ref[...] = z * jnp.dot(w, bst_ref[...],
                             preferred_element_type=jnp.float32)


_fused = pl.pallas_call(
    _fused_body,
    grid=(_MPAD // 512,),
    in_specs=[
        pl.BlockSpec((512, _D), lambda i: (i, 0)),
        pl.BlockSpec((512, _D), lambda i: (i, 0)),
        pl.BlockSpec((_D, _K), lambda i: (0, 0)),
        pl.BlockSpec((_K, _D), lambda i: (0, 0)),
    ],
    out_specs=pl.BlockSpec((512, _D), lambda i: (i, 0)),
    out_shape=jax.ShapeDtypeStruct((_MPAD, _D), jnp.float32),
)


def _pca_body(f_ref, w_ref, b_ref, bs_ref, bst_ref, o_ref):
    x = jnp.dot(f_ref[...], w_ref[...], preferred_element_type=jnp.float32)
    x = jnp.maximum(x + b_ref[...], 0.0)
    ss = jnp.dot(x * x, bs_ref[...], preferred_element_type=jnp.float32)
    rs = lax.rsqrt(jnp.maximum(ss, 1e-24))
    o_ref[...] = x * jnp.dot(rs, bst_ref[...],
                             preferred_element_type=jnp.float32)


_pca = pl.pallas_call(
    _pca_body,
    grid=(_NROWS // 256,),
    in_specs=[
        pl.BlockSpec((256, _D), lambda i: (i, 0)),
        pl.BlockSpec((_D, _D), lambda i: (0, 0)),
        pl.BlockSpec((1, _D), lambda i: (0, 0)),
        pl.BlockSpec((_D, _K), lambda i: (0, 0)),
        pl.BlockSpec((_K, _D), lambda i: (0, 0)),
    ],
    out_specs=pl.BlockSpec((256, _D), lambda i: (i, 0)),
    out_shape=jax.ShapeDtypeStruct((_NROWS, _D), jnp.float32),
)


def _mlp_body(x_ref, w_ref, b_ref, o_ref):
    l = jnp.dot(x_ref[...], w_ref[...], preferred_element_type=jnp.float32)
    l = l + b_ref[...]
    m = jnp.max(l, axis=1, keepdims=True)
    e = jnp.exp(l - m)
    o_ref[...] = e / jnp.sum(e, axis=1, keepdims=True)


_mlp = pl.pallas_call(
    _mlp_body,
    grid=(_NROWS // 256,),
    in_specs=[
        pl.BlockSpec((256, _D), lambda i: (i, 0)),
        pl.BlockSpec((_D, _D), lambda i: (0, 0)),
        pl.BlockSpec((1, _D), lambda i: (0, 0)),
    ],
    out_specs=pl.BlockSpec((256, _D), lambda i: (i, 0)),
    out_shape=jax.ShapeDtypeStruct((_NROWS, _D), jnp.float32),
)


@jax.jit
def _run(feat, src, trg, pca_w, pca_b, mlp_w, mlp_b):
    featp = jnp.zeros((_NROWS, _D), jnp.float32).at[:_N].set(feat)
    srcp = jnp.concatenate(
        [src.astype(jnp.int32), jnp.zeros((_MPAD - _M,), jnp.int32)])
    trgp = jnp.concatenate(
        [trg.astype(jnp.int32), jnp.full((_MPAD - _M,), _SENT, jnp.int32)])
    cap = jnp.arange(_D, dtype=jnp.int32) // _DD
    bs = (cap[:, None] == jnp.arange(_K)[None, :]).astype(jnp.float32)
    wp = jnp.zeros((_D, _D), jnp.float32).at[:, :_NCLASS].set(mlp_w)
    bp = jnp.full((_D,), -1e30, jnp.float32).at[:_NCLASS].set(mlp_b)

    x = _pca(featp, pca_w, pca_b.reshape(1, _D), bs, bs.T)
    for _layer in range(_NLAYER):
        zg = _gath(x, srcp)
        cc = x
        for _it in range(_ROUTIT):
            cg = _gath(cc, trgp)
            wz = _fused(zg, cg, bs, bs.T)
            cc = _scat(wz, trgp, x)
        x = cc
    probs = _mlp(x, wp, bp.reshape(1, _D))
    return probs[:_N, :_NCLASS]


def kernel(feat, src_trg_edges, pca_w, pca_b, mlp_w, mlp_b):
    return _run(feat, src_trg_edges[0], src_trg_edges[1],
                pca_w, pca_b, mlp_w, mlp_b)


# speedup vs baseline: 4.9248x; 1.1582x over previous
import functools

import jax
import jax.numpy as jnp
from jax import lax
from jax.experimental import pallas as pl
from jax.experimental.pallas import tpu as pltpu
from jax.experimental.pallas import tpu_sc as plsc

_N = 10000
_D = 128
_K = 8
_DD = 16
_ROUTIT = 6
_NLAYER = 3
_NCLASS = 40

_NROWS = 10496
_HALF = 5120
_SENT = 10240
_M = 320000
_CH = 128
_NT = 16
_MPAD = 327680
_NCHUNK_G = _MPAD // (_CH * 2 * _NT)
_NCHUNK_S = _MPAD // (_CH * _NT)
_GROWS = _MPAD // 32
_AGG_ROWS = _HALF + 64
_ROWS_PT = _HALF // _NT

_SC_PARAMS = pltpu.CompilerParams(needs_layout_passes=False)
_SC_MESH = plsc.VectorSubcoreMesh(core_axis_name="c", subcore_axis_name="s")


def _rsqrt_nr(ss):
    bi = plsc.bitcast(ss, jnp.int32)
    y = plsc.bitcast(jnp.int32(0x5F3759DF) - (bi >> 1), jnp.float32)
    y = y * (1.5 - 0.5 * ss * y * y)
    y = y * (1.5 - 0.5 * ss * y * y)
    y = y * (1.5 - 0.5 * ss * y * y)
    return y


def _gath_body(tab, idx, out, iball, b0, b1, b2, b3, gsem, osem):
    wid = lax.axis_index("s") * 2 + lax.axis_index("c")
    tlo = wid * _GROWS
    pltpu.sync_copy(idx.at[pl.ds(tlo, _GROWS)], iball)
    bufs = (b0, b1, b2, b3)

    def _quad(i, _):
        ds = []
        for b in range(4):
            base = (4 * i + b) * _CH
            ds.append(pltpu.async_copy(
                tab.at[iball.at[pl.ds(base, _CH)]], bufs[b], gsem))
        for b in range(4):
            ds[b].wait()
        os = []
        for b in range(4):
            base = (4 * i + b) * _CH
            os.append(pltpu.async_copy(
                bufs[b], out.at[pl.ds(tlo + base, _CH)], osem))
        for b in range(4):
            os[b].wait()
        return 0
    lax.fori_loop(0, _NCHUNK_G // 4, _quad, 0)


_gath = pl.kernel(
    _gath_body,
    out_type=jax.ShapeDtypeStruct((_MPAD, _D), jnp.float32),
    mesh=_SC_MESH,
    compiler_params=_SC_PARAMS,
    scratch_types=[
        pltpu.VMEM((_GROWS,), jnp.int32),
        pltpu.VMEM((_CH, _D), jnp.float32),
        pltpu.VMEM((_CH, _D), jnp.float32),
        pltpu.VMEM((_CH, _D), jnp.float32),
        pltpu.VMEM((_CH, _D), jnp.float32),
        pltpu.SemaphoreType.DMA,
        pltpu.SemaphoreType.DMA,
    ],
)


def _scat_body(wz, trgp, xn, c_new, zbuf, zbuf1, cbuf, tgall, iloc, iloc1,
               agg, sem, ssem):
    cid = lax.axis_index("c")
    sid = lax.axis_index("s")
    my_lo = cid * _HALF
    lane = lax.iota(jnp.int32, _DD)
    cols = [jnp.full((_DD,), kj, jnp.int32) for kj in range(_D)]

    def _zero_zbuf(i, _):
        for k in range(_K):
            zbuf[i, pl.ds(k * _DD, _DD)] = jnp.zeros((_DD,), jnp.float32)
        return 0
    lax.fori_loop(0, _CH, _zero_zbuf, 0)

    rpt = _AGG_ROWS // _NT
    abase = sid * rpt
    pltpu.sync_copy(zbuf, agg.at[pl.ds(abase, _CH)])
    pltpu.sync_copy(zbuf, agg.at[pl.ds(abase + _CH, _CH)])
    pltpu.sync_copy(zbuf.at[pl.ds(0, rpt - 2 * _CH)],
                    agg.at[pl.ds(abase + 2 * _CH, rpt - 2 * _CH)])
    pltpu.sync_copy(trgp.at[pl.ds(sid * _NCHUNK_S * _CH, _NCHUNK_S * _CH)],
                    tgall)
    plsc.subcore_barrier()

    zbs = (zbuf, zbuf1)
    ils = (iloc, iloc1)

    def _pair(i, _):
        ds = []
        for b in range(2):
            base = (sid * _NCHUNK_S + 2 * i + b) * _CH
            ds.append(pltpu.async_copy(wz.at[pl.ds(base, _CH)], zbs[b], sem))
        for b in range(2):
            ch = 2 * i + b
            for g in range(_CH // _DD):
                sl = pl.ds(g * _DD, _DD)
                loc = tgall[pl.ds(ch * _CH + g * _DD, _DD)] - my_lo
                bad = (loc < 0) | (loc >= _HALF)
                ils[b][sl] = jnp.where(bad, _HALF, loc)
        for b in range(2):
            ds[b].wait()
        ss = []
        for b in range(2):
            ss.append(pltpu.async_copy(zbs[b], agg.at[ils[b]], ssem,
                                       add=True))
        for b in range(2):
            ss[b].wait()
        return 0
    lax.fori_loop(0, _NCHUNK_S // 2, _pair, 0)
    plsc.subcore_barrier()

    nb = sid * _ROWS_PT
    for off, nr in ((0, _CH), (_CH, _CH), (2 * _CH, _ROWS_PT - 2 * _CH)):
        pltpu.sync_copy(agg.at[pl.ds(nb + off, nr)], cbuf.at[pl.ds(0, nr)])
        pltpu.sync_copy(xn.at[pl.ds(my_lo + nb + off, nr)],
                        zbuf.at[pl.ds(0, nr)])

        def _fin(g, _):
            rows = g * _DD + lane
            for k in range(_K):
                acc = None
                tk = []
                for j in range(_DD):
                    col = cols[k * _DD + j]
                    t = (plsc.load_gather(zbuf, [rows, col]) +
                         plsc.load_gather(cbuf, [rows, col]))
                    tk.append(t)
                    acc = t * t if acc is None else acc + t * t
                y = _rsqrt_nr(jnp.maximum(acc, 1e-24))
                for j in range(_DD):
                    plsc.store_scatter(zbuf, [rows, cols[k * _DD + j]],
                                       tk[j] * y)
            return 0
        lax.fori_loop(0, nr // _DD, _fin, 0)
        pltpu.sync_copy(zbuf.at[pl.ds(0, nr)],
                        c_new.at[pl.ds(my_lo + nb + off, nr)])


_scat = pl.kernel(
    _scat_body,
    out_type=jax.ShapeDtypeStruct((_NROWS, _D), jnp.float32),
    mesh=_SC_MESH,
    compiler_params=_SC_PARAMS,
    scratch_types=[
        pltpu.VMEM((_CH, _D), jnp.float32),
        pltpu.VMEM((_CH, _D), jnp.float32),
        pltpu.VMEM((_CH, _D), jnp.float32),
        pltpu.VMEM((_NCHUNK_S * _CH,), jnp.int32),
        pltpu.VMEM((_CH,), jnp.int32),
        pltpu.VMEM((_CH,), jnp.int32),
        pltpu.VMEM_SHARED((_AGG_ROWS, _D), jnp.float32),
        pltpu.SemaphoreType.DMA,
        pltpu.SemaphoreType.DMA,
    ],
)


def _fused_body(z_ref, cg_ref, bs_ref, bst_ref, o_ref):
    z = z_ref[...]
    p = jnp.dot(z * cg_ref[...], bs_ref[...],
                preferred_element_type=jnp.float32)
    m = jnp.max(p, axis=1, keepdims=True)
    e = jnp.exp(p - m)
    w = e / jnp.sum(e, axis=1, keepdims=True)
    o_ref[...] = z * jnp.dot(w, bst_ref[...],
                             preferred_element_type=jnp.float32)


_fused = pl.pallas_call(
    _fused_body,
    grid=(_MPAD // 2048,),
    in_specs=[
        pl.BlockSpec((2048, _D), lambda i: (i, 0)),
        pl.BlockSpec((2048, _D), lambda i: (i, 0)),
        pl.BlockSpec((_D, _K), lambda i: (0, 0)),
        pl.BlockSpec((_K, _D), lambda i: (0, 0)),
    ],
    out_specs=pl.BlockSpec((2048, _D), lambda i: (i, 0)),
    out_shape=jax.ShapeDtypeStruct((_MPAD, _D), jnp.float32),
)


def _pca_body(f_ref, w_ref, b_ref, bs_ref, bst_ref, o_ref):
    x = jnp.dot(f_ref[...], w_ref[...], preferred_element_type=jnp.float32)
    x = jnp.maximum(x + b_ref[...], 0.0)
    ss = jnp.dot(x * x, bs_ref[...], preferred_element_type=jnp.float32)
    rs = lax.rsqrt(jnp.maximum(ss, 1e-24))
    o_ref[...] = x * jnp.dot(rs, bst_ref[...],
                             preferred_element_type=jnp.float32)


_pca = pl.pallas_call(
    _pca_body,
    grid=(_NROWS // 256,),
    in_specs=[
        pl.BlockSpec((256, _D), lambda i: (i, 0)),
        pl.BlockSpec((_D, _D), lambda i: (0, 0)),
        pl.BlockSpec((1, _D), lambda i: (0, 0)),
        pl.BlockSpec((_D, _K), lambda i: (0, 0)),
        pl.BlockSpec((_K, _D), lambda i: (0, 0)),
    ],
    out_specs=pl.BlockSpec((256, _D), lambda i: (i, 0)),
    out_shape=jax.ShapeDtypeStruct((_NROWS, _D), jnp.float32),
)


def _mlp_body(x_ref, w_ref, b_ref, o_ref):
    l = jnp.dot(x_ref[...], w_ref[...], preferred_element_type=jnp.float32)
    l = l + b_ref[...]
    m = jnp.max(l, axis=1, keepdims=True)
    e = jnp.exp(l - m)
    o_ref[...] = e / jnp.sum(e, axis=1, keepdims=True)


_mlp = pl.pallas_call(
    _mlp_body,
    grid=(_NROWS // 256,),
    in_specs=[
        pl.BlockSpec((256, _D), lambda i: (i, 0)),
        pl.BlockSpec((_D, _D), lambda i: (0, 0)),
        pl.BlockSpec((1, _D), lambda i: (0, 0)),
    ],
    out_specs=pl.BlockSpec((256, _D), lambda i: (i, 0)),
    out_shape=jax.ShapeDtypeStruct((_NROWS, _D), jnp.float32),
)


@jax.jit
def _run(feat, src, trg, pca_w, pca_b, mlp_w, mlp_b):
    featp = jnp.zeros((_NROWS, _D), jnp.float32).at[:_N].set(feat)
    srcp = jnp.concatenate(
        [src.astype(jnp.int32), jnp.zeros((_MPAD - _M,), jnp.int32)])
    trgp = jnp.concatenate(
        [trg.astype(jnp.int32), jnp.full((_MPAD - _M,), _SENT, jnp.int32)])
    cap = jnp.arange(_D, dtype=jnp.int32) // _DD
    bs = (cap[:, None] == jnp.arange(_K)[None, :]).astype(jnp.float32)
    wp = jnp.zeros((_D, _D), jnp.float32).at[:, :_NCLASS].set(mlp_w)
    bp = jnp.full((_D,), -1e30, jnp.float32).at[:_NCLASS].set(mlp_b)

    x = _pca(featp, pca_w, pca_b.reshape(1, _D), bs, bs.T)
    for _layer in range(_NLAYER):
        zg = _gath(x, srcp)
        cc = x
        for _it in range(_ROUTIT):
            cg = _gath(cc, trgp)
            wz = _fused(zg, cg, bs, bs.T)
            cc = _scat(wz, trgp, x)
        x = cc
    probs = _mlp(x, wp, bp.reshape(1, _D))
    return probs[:_N, :_NCLASS]


def kernel(feat, src_trg_edges, pca_w, pca_b, mlp_w, mlp_b):
    return _run(feat, src_trg_edges[0], src_trg_edges[1],
                pca_w, pca_b, mlp_w, mlp_b)


# speedup vs baseline: 5.0153x; 1.0184x over previous
import functools

import jax
import jax.numpy as jnp
from jax import lax
from jax.experimental import pallas as pl
from jax.experimental.pallas import tpu as pltpu
from jax.experimental.pallas import tpu_sc as plsc

_N = 10000
_D = 128
_K = 8
_DD = 16
_ROUTIT = 6
_NLAYER = 3
_NCLASS = 40

_NROWS = 10496
_HALF = 5120
_SENT = 10240
_M = 320000
_CH = 128
_NT = 16
_MPAD = 327680
_NCHUNK_G = _MPAD // (_CH * 2 * _NT)
_NCHUNK_S = _MPAD // (_CH * _NT)
_GROWS = _MPAD // 32
_AGG_ROWS = _HALF + 64
_ROWS_PT = _HALF // _NT

_SC_PARAMS = pltpu.CompilerParams(needs_layout_passes=False)
_SC_MESH = plsc.VectorSubcoreMesh(core_axis_name="c", subcore_axis_name="s")


def _rsqrt_nr(ss):
    bi = plsc.bitcast(ss, jnp.int32)
    y = plsc.bitcast(jnp.int32(0x5F3759DF) - (bi >> 1), jnp.float32)
    y = y * (1.5 - 0.5 * ss * y * y)
    y = y * (1.5 - 0.5 * ss * y * y)
    y = y * (1.5 - 0.5 * ss * y * y)
    return y


_Q0 = 120
_Q1 = 40


def _gath_body(tab, idx, out, iball, b0, b1, b2, b3, gsem, osem):
    cid = lax.axis_index("c")
    sid = lax.axis_index("s")
    q = jnp.where(cid == 0, _Q0, _Q1)
    tlo = jnp.where(cid == 0, sid * _Q0, 16 * _Q0 + sid * _Q1) * _CH

    @pl.when(cid == 0)
    def _():
        pltpu.sync_copy(idx.at[pl.ds(tlo, _Q0 * _CH)], iball)

    @pl.when(cid != 0)
    def _():
        pltpu.sync_copy(idx.at[pl.ds(tlo, _Q1 * _CH)],
                        iball.at[pl.ds(0, _Q1 * _CH)])
    bufs = (b0, b1, b2, b3)

    def _quad(i, _):
        ds = []
        for b in range(4):
            base = (4 * i + b) * _CH
            ds.append(pltpu.async_copy(
                tab.at[iball.at[pl.ds(base, _CH)]], bufs[b], gsem))
        for b in range(4):
            ds[b].wait()
        os = []
        for b in range(4):
            base = (4 * i + b) * _CH
            os.append(pltpu.async_copy(
                bufs[b], out.at[pl.ds(tlo + base, _CH)], osem))
        for b in range(4):
            os[b].wait()
        return 0
    lax.fori_loop(0, q // 4, _quad, 0)


_gath = pl.kernel(
    _gath_body,
    out_type=jax.ShapeDtypeStruct((_MPAD, _D), jnp.float32),
    mesh=_SC_MESH,
    compiler_params=_SC_PARAMS,
    scratch_types=[
        pltpu.VMEM((_Q0 * _CH,), jnp.int32),
        pltpu.VMEM((_CH, _D), jnp.float32),
        pltpu.VMEM((_CH, _D), jnp.float32),
        pltpu.VMEM((_CH, _D), jnp.float32),
        pltpu.VMEM((_CH, _D), jnp.float32),
        pltpu.SemaphoreType.DMA,
        pltpu.SemaphoreType.DMA,
    ],
)


def _scat_body(wz, trgp, xn, c_new, zbuf, zbuf1, cbuf, tgall, iloc, iloc1,
               agg, sem, ssem):
    cid = lax.axis_index("c")
    sid = lax.axis_index("s")
    my_lo = cid * _HALF
    lane = lax.iota(jnp.int32, _DD)
    cols = [jnp.full((_DD,), kj, jnp.int32) for kj in range(_D)]

    def _zero_zbuf(i, _):
        for k in range(_K):
            zbuf[i, pl.ds(k * _DD, _DD)] = jnp.zeros((_DD,), jnp.float32)
        return 0
    lax.fori_loop(0, _CH, _zero_zbuf, 0)

    rpt = _AGG_ROWS // _NT
    abase = sid * rpt
    pltpu.sync_copy(zbuf, agg.at[pl.ds(abase, _CH)])
    pltpu.sync_copy(zbuf, agg.at[pl.ds(abase + _CH, _CH)])
    pltpu.sync_copy(zbuf.at[pl.ds(0, rpt - 2 * _CH)],
                    agg.at[pl.ds(abase + 2 * _CH, rpt - 2 * _CH)])
    pltpu.sync_copy(trgp.at[pl.ds(sid * _NCHUNK_S * _CH, _NCHUNK_S * _CH)],
                    tgall)
    plsc.subcore_barrier()

    zbs = (zbuf, zbuf1)
    ils = (iloc, iloc1)

    def _pair(i, _):
        ds = []
        for b in range(2):
            base = (sid * _NCHUNK_S + 2 * i + b) * _CH
            ds.append(pltpu.async_copy(wz.at[pl.ds(base, _CH)], zbs[b], sem))
        for b in range(2):
            ch = 2 * i + b
            for g in range(_CH // _DD):
                sl = pl.ds(g * _DD, _DD)
                loc = tgall[pl.ds(ch * _CH + g * _DD, _DD)] - my_lo
                bad = (loc < 0) | (loc >= _HALF)
                ils[b][sl] = jnp.where(bad, _HALF, loc)
        for b in range(2):
            ds[b].wait()
        ss = []
        for b in range(2):
            ss.append(pltpu.async_copy(zbs[b], agg.at[ils[b]], ssem,
                                       add=True))
        for b in range(2):
            ss[b].wait()
        return 0
    lax.fori_loop(0, _NCHUNK_S // 2, _pair, 0)
    plsc.subcore_barrier()

    nb = sid * _ROWS_PT
    for off, nr in ((0, _CH), (_CH, _CH), (2 * _CH, _ROWS_PT - 2 * _CH)):
        pltpu.sync_copy(agg.at[pl.ds(nb + off, nr)], cbuf.at[pl.ds(0, nr)])
        pltpu.sync_copy(xn.at[pl.ds(my_lo + nb + off, nr)],
                        zbuf.at[pl.ds(0, nr)])

        def _fin(g, _):
            rows = g * _DD + lane
            for k in range(_K):
                acc = None
                tk = []
                for j in range(_DD):
                    col = cols[k * _DD + j]
                    t = (plsc.load_gather(zbuf, [rows, col]) +
                         plsc.load_gather(cbuf, [rows, col]))
                    tk.append(t)
                    acc = t * t if acc is None else acc + t * t
                y = _rsqrt_nr(jnp.maximum(acc, 1e-24))
                for j in range(_DD):
                    plsc.store_scatter(zbuf, [rows, cols[k * _DD + j]],
                                       tk[j] * y)
            return 0
        lax.fori_loop(0, nr // _DD, _fin, 0)
        pltpu.sync_copy(zbuf.at[pl.ds(0, nr)],
                        c_new.at[pl.ds(my_lo + nb + off, nr)])


_scat = pl.kernel(
    _scat_body,
    out_type=jax.ShapeDtypeStruct((_NROWS, _D), jnp.float32),
    mesh=_SC_MESH,
    compiler_params=_SC_PARAMS,
    scratch_types=[
        pltpu.VMEM((_CH, _D), jnp.float32),
        pltpu.VMEM((_CH, _D), jnp.float32),
        pltpu.VMEM((_CH, _D), jnp.float32),
        pltpu.VMEM((_NCHUNK_S * _CH,), jnp.int32),
        pltpu.VMEM((_CH,), jnp.int32),
        pltpu.VMEM((_CH,), jnp.int32),
        pltpu.VMEM_SHARED((_AGG_ROWS, _D), jnp.float32),
        pltpu.SemaphoreType.DMA,
        pltpu.SemaphoreType.DMA,
    ],
)


def _fused_body(z_ref, cg_ref, bs_ref, bst_ref, o_ref):
    z = z_ref[...]
    p = jnp.dot(z * cg_ref[...], bs_ref[...],
                preferred_element_type=jnp.float32)
    m = jnp.max(p, axis=1, keepdims=True)
    e = jnp.exp(p - m)
    w = e / jnp.sum(e, axis=1, keepdims=True)
    o_ref[...] = z * jnp.dot(w, bst_ref[...],
                             preferred_element_type=jnp.float32)


_fused = pl.pallas_call(
    _fused_body,
    grid=(_MPAD // 2048,),
    in_specs=[
        pl.BlockSpec((2048, _D), lambda i: (i, 0)),
        pl.BlockSpec((2048, _D), lambda i: (i, 0)),
        pl.BlockSpec((_D, _K), lambda i: (0, 0)),
        pl.BlockSpec((_K, _D), lambda i: (0, 0)),
    ],
    out_specs=pl.BlockSpec((2048, _D), lambda i: (i, 0)),
    out_shape=jax.ShapeDtypeStruct((_MPAD, _D), jnp.float32),
)


def _pca_body(f_ref, w_ref, b_ref, bs_ref, bst_ref, o_ref):
    x = jnp.dot(f_ref[...], w_ref[...], preferred_element_type=jnp.float32)
    x = jnp.maximum(x + b_ref[...], 0.0)
    ss = jnp.dot(x * x, bs_ref[...], preferred_element_type=jnp.float32)
    rs = lax.rsqrt(jnp.maximum(ss, 1e-24))
    o_ref[...] = x * jnp.dot(rs, bst_ref[...],
                             preferred_element_type=jnp.float32)


_pca = pl.pallas_call(
    _pca_body,
    grid=(_NROWS // 256,),
    in_specs=[
        pl.BlockSpec((256, _D), lambda i: (i, 0)),
        pl.BlockSpec((_D, _D), lambda i: (0, 0)),
        pl.BlockSpec((1, _D), lambda i: (0, 0)),
        pl.BlockSpec((_D, _K), lambda i: (0, 0)),
        pl.BlockSpec((_K, _D), lambda i: (0, 0)),
    ],
    out_specs=pl.BlockSpec((256, _D), lambda i: (i, 0)),
    out_shape=jax.ShapeDtypeStruct((_NROWS, _D), jnp.float32),
)


def _mlp_body(x_ref, w_ref, b_ref, o_ref):
    l = jnp.dot(x_ref[...], w_ref[...], preferred_element_type=jnp.float32)
    l = l + b_ref[...]
    m = jnp.max(l, axis=1, keepdims=True)
    e = jnp.exp(l - m)
    o_ref[...] = e / jnp.sum(e, axis=1, keepdims=True)


_mlp = pl.pallas_call(
    _mlp_body,
    grid=(_NROWS // 256,),
    in_specs=[
        pl.BlockSpec((256, _D), lambda i: (i, 0)),
        pl.BlockSpec((_D, _D), lambda i: (0, 0)),
        pl.BlockSpec((1, _D), lambda i: (0, 0)),
    ],
    out_specs=pl.BlockSpec((256, _D), lambda i: (i, 0)),
    out_shape=jax.ShapeDtypeStruct((_NROWS, _D), jnp.float32),
)


@jax.jit
def _run(feat, src, trg, pca_w, pca_b, mlp_w, mlp_b):
    featp = jnp.zeros((_NROWS, _D), jnp.float32).at[:_N].set(feat)
    srcp = jnp.concatenate(
        [src.astype(jnp.int32), jnp.zeros((_MPAD - _M,), jnp.int32)])
    trgp = jnp.concatenate(
        [trg.astype(jnp.int32), jnp.full((_MPAD - _M,), _SENT, jnp.int32)])
    cap = jnp.arange(_D, dtype=jnp.int32) // _DD
    bs = (cap[:, None] == jnp.arange(_K)[None, :]).astype(jnp.float32)
    wp = jnp.zeros((_D, _D), jnp.float32).at[:, :_NCLASS].set(mlp_w)
    bp = jnp.full((_D,), -1e30, jnp.float32).at[:_NCLASS].set(mlp_b)

    x = _pca(featp, pca_w, pca_b.reshape(1, _D), bs, bs.T)
    for _layer in range(_NLAYER):
        zg = _gath(x, srcp)
        cc = x
        for _it in range(_ROUTIT):
            cg = _gath(cc, trgp)
            wz = _fused(zg, cg, bs, bs.T)
            cc = _scat(wz, trgp, x)
        x = cc
    probs = _mlp(x, wp, bp.reshape(1, _D))
    return probs[:_N, :_NCLASS]


def kernel(feat, src_trg_edges, pca_w, pca_b, mlp_w, mlp_b):
    return _run(feat, src_trg_edges[0], src_trg_edges[1],
                pca_w, pca_b, mlp_w, mlp_b)
